# Initial kernel scaffold; baseline (speedup 1.0000x reference)
import jax
import jax.numpy as jnp
from jax.experimental import pallas as pl


def kernel(adj_matrix, node_features, current_node_idx, W1, att_src1, att_dst1, lin_edge1, att_edge1, b1, W2, att_src2, att_dst2, lin_edge2, att_edge2, b2, Wp, bp, Wv, bv):
    raise NotImplementedError("write your pallas kernel here")


# speedup vs baseline: 1759.4002x; 1759.4002x over previous
import functools

import jax
import jax.numpy as jnp
from jax import lax
from jax.experimental import pallas as pl
from jax.experimental.pallas import tpu as pltpu

N = 1024
HID = 32
HI = lax.Precision.HIGHEST
F32 = jnp.float32


def _leaky(x):
    return jnp.where(x >= 0, x, 0.2 * x)


def _proj_body(adjT_ref, x_ref, w_ref, asrc_ref, adst_ref, le_ref, ae_ref,
               h_ref, asp_ref, adc_ref, tc_ref, mu_ref, *, heads):
    adjT = adjT_ref[...]
    ones_col = jnp.ones((N, 1), F32)
    ssum = jnp.sum(jnp.dot(adjT, ones_col, precision=HI,
                           preferred_element_type=F32))
    csum = jnp.sum(jnp.dot(jnp.where(adjT != 0, 1.0, 0.0).astype(F32),
                           ones_col, precision=HI, preferred_element_type=F32))
    mu = ssum / csum
    mu_ref[0, 0] = mu
    h = jnp.dot(x_ref[...], w_ref[...], precision=HI,
                preferred_element_type=F32)
    for hd in range(heads):
        hh = h[:, hd * HID:(hd + 1) * HID]
        h_ref[hd] = hh
        asv = asrc_ref[hd:hd + 1, :]
        adv = adst_ref[hd:hd + 1, :]
        ce = jnp.sum(le_ref[0:1, hd * HID:(hd + 1) * HID]
                     * ae_ref[0:1, hd * HID:(hd + 1) * HID])
        dn = (((1,), (1,)), ((), ()))
        as_col = lax.dot_general(hh, asv, dn, precision=HI,
                                 preferred_element_type=F32)
        ad_col = lax.dot_general(hh, adv, dn, precision=HI,
                                 preferred_element_type=F32)
        as_row = lax.dot_general(asv, hh, dn, precision=HI,
                                 preferred_element_type=F32)
        asp_ref[hd] = as_row + ce
        adc_ref[hd] = ad_col
        tc_ref[hd] = _leaky(as_col + ad_col + mu * ce)


def _attn_body(adjT_ref, h_ref, asp_ref, adc_ref, tc_ref, b_ref, out_ref,
               *, bj, do_relu):
    s = adc_ref[0] + asp_ref[0]
    s = _leaky(s)
    sm = jnp.where(adjT_ref[...] != 0, s, -1e30)
    t = tc_ref[0]
    m = jnp.maximum(jnp.max(sm, axis=1, keepdims=True), t)
    e = jnp.exp(sm - m)
    esl = jnp.exp(t - m)
    rows = lax.broadcasted_iota(jnp.int32, (bj, N), 0) + pl.program_id(0) * bj
    cols = lax.broadcasted_iota(jnp.int32, (bj, N), 1)
    e = e + jnp.where(rows == cols, esl, 0.0)
    den = jnp.sum(e, axis=1, keepdims=True) + 1e-16
    p = e * (1.0 / den)
    out = jnp.dot(p, h_ref[0], precision=HI, preferred_element_type=F32)
    out = out + b_ref[0]
    if do_relu:
        out = jnp.maximum(out, 0.0)
    out_ref[0] = out


def _head_body(emb_ref, idx_ref, wp_ref, bp_ref, wv_ref, bv_ref,
               probs_ref, sv_ref):
    emb = emb_ref[...]
    g = jnp.mean(emb, axis=0, keepdims=True)
    sv_ref[...] = jnp.dot(g, wv_ref[...], precision=HI,
                          preferred_element_type=F32) + bv_ref[...]
    idx = idx_ref[0, 0]
    rowsel = lax.broadcasted_iota(jnp.int32, (N, 1), 0) == idx
    sel = jnp.sum(jnp.where(rowsel, emb, 0.0), axis=0, keepdims=True)
    logits = jnp.dot(sel, wp_ref[...], precision=HI,
                     preferred_element_type=F32) + bp_ref[...]
    mx = jnp.max(logits, axis=1, keepdims=True)
    ex = jnp.exp(logits - mx)
    probs_ref[...] = ex / jnp.sum(ex, axis=1, keepdims=True)


def _run_proj(adjT, x, w, asrc, adst, le, ae, heads):
    body = functools.partial(_proj_body, heads=heads)
    return pl.pallas_call(
        body,
        out_shape=[
            jax.ShapeDtypeStruct((heads, N, HID), F32),
            jax.ShapeDtypeStruct((heads, 1, N), F32),
            jax.ShapeDtypeStruct((heads, N, 1), F32),
            jax.ShapeDtypeStruct((heads, N, 1), F32),
            jax.ShapeDtypeStruct((1, 1), F32),
        ],
        out_specs=[
            pl.BlockSpec(), pl.BlockSpec(), pl.BlockSpec(), pl.BlockSpec(),
            pl.BlockSpec(memory_space=pltpu.SMEM),
        ],
    )(adjT, x, w, asrc, adst, le, ae)


def _run_attn(adjT, h, asp, adc, tc, b, heads, bj, do_relu):
    nj = N // bj
    body = functools.partial(_attn_body, bj=bj, do_relu=do_relu)
    return pl.pallas_call(
        body,
        grid=(nj, heads),
        in_specs=[
            pl.BlockSpec((bj, N), lambda j, hd: (j, 0)),
            pl.BlockSpec((1, N, HID), lambda j, hd: (hd, 0, 0)),
            pl.BlockSpec((1, 1, N), lambda j, hd: (hd, 0, 0)),
            pl.BlockSpec((1, bj, 1), lambda j, hd: (hd, j, 0)),
            pl.BlockSpec((1, bj, 1), lambda j, hd: (hd, j, 0)),
            pl.BlockSpec((1, 1, HID), lambda j, hd: (hd, 0, 0)),
        ],
        out_specs=pl.BlockSpec((1, bj, HID), lambda j, hd: (hd, j, 0)),
        out_shape=jax.ShapeDtypeStruct((heads, N, HID), F32),
    )(adjT, h, asp, adc, tc, b)


def kernel(adj_matrix, node_features, current_node_idx, W1, att_src1,
           att_dst1, lin_edge1, att_edge1, b1, W2, att_src2, att_dst2,
           lin_edge2, att_edge2, b2, Wp, bp, Wv, bv):
    adjT = adj_matrix.T
    idx = jnp.asarray(current_node_idx, jnp.int32).reshape(1, 1)

    h1, asp1, adc1, tc1, mu = _run_proj(
        adjT, node_features, W1, att_src1.reshape(4, HID),
        att_dst1.reshape(4, HID), lin_edge1, att_edge1.reshape(1, 4 * HID),
        heads=4)
    out1 = _run_attn(adjT, h1, asp1, adc1, tc1, b1.reshape(4, 1, HID),
                     heads=4, bj=256, do_relu=True)
    h_mid = jnp.transpose(out1, (1, 0, 2)).reshape(N, 4 * HID)

    h2, asp2, adc2, tc2, _ = _run_proj(
        adjT, h_mid, W2, att_src2.reshape(1, HID), att_dst2.reshape(1, HID),
        lin_edge2, att_edge2.reshape(1, HID), heads=1)
    out2 = _run_attn(adjT, h2, asp2, adc2, tc2, b2.reshape(1, 1, HID),
                     heads=1, bj=256, do_relu=False)
    emb = out2.reshape(N, HID)

    probs, sv = pl.pallas_call(
        _head_body,
        in_specs=[
            pl.BlockSpec(),
            pl.BlockSpec(memory_space=pltpu.SMEM),
            pl.BlockSpec(), pl.BlockSpec(), pl.BlockSpec(), pl.BlockSpec(),
        ],
        out_shape=[jax.ShapeDtypeStruct((1, 2), F32),
                   jax.ShapeDtypeStruct((1, 1), F32)],
    )(emb, idx, Wp, bp.reshape(1, 2), Wv, bv.reshape(1, 1))
    return probs.reshape(2), sv.reshape(1)
